# baseline (speedup 1.0000x reference)
import functools

import jax
import jax.numpy as jnp
from jax import lax
from jax.experimental import pallas as pl
from jax.experimental.pallas import tpu as pltpu
from jax.experimental.pallas import tpu_sc as plsc

_info = plsc.get_sparse_core_info()
_NC, _NS, _L = _info.num_cores, _info.num_subcores, _info.num_lanes
_NW = _NC * _NS


def _make_sc_score(batch, dim):
  half = dim // 2
  chunks = half // _L
  bpw = batch // _NW
  mesh = plsc.VectorSubcoreMesh(core_axis_name="c", subcore_axis_name="s")

  @functools.partial(
      pl.kernel,
      mesh=mesh,
      out_type=jax.ShapeDtypeStruct((batch,), jnp.float32),
      compiler_params=pltpu.CompilerParams(needs_layout_passes=False),
      scratch_types=[
          pltpu.VMEM((bpw,), jnp.int32),
          pltpu.VMEM((bpw,), jnp.int32),
          pltpu.VMEM((bpw,), jnp.int32),
          pltpu.VMEM((bpw, dim), jnp.float32),
          pltpu.VMEM((bpw, dim), jnp.float32),
          pltpu.VMEM((bpw, dim), jnp.float32),
          pltpu.VMEM((bpw,), jnp.float32),
          pltpu.SemaphoreType.DMA,
      ],
  )
  def sc_score(hidx_hbm, ridx_hbm, tidx_hbm, ent_hbm, rel_hbm, out_hbm,
               hidx_v, ridx_v, tidx_v, hrow_v, rrow_v, trow_v, out_v, sem):
    wid = lax.axis_index("s") * _NC + lax.axis_index("c")
    base = wid * bpw
    pltpu.sync_copy(hidx_hbm.at[pl.ds(base, bpw)], hidx_v)
    pltpu.sync_copy(ridx_hbm.at[pl.ds(base, bpw)], ridx_v)
    pltpu.sync_copy(tidx_hbm.at[pl.ds(base, bpw)], tidx_v)
    ch = pltpu.async_copy(ent_hbm.at[hidx_v], hrow_v, sem)
    cr = pltpu.async_copy(rel_hbm.at[ridx_v], rrow_v, sem)
    ct = pltpu.async_copy(ent_hbm.at[tidx_v], trow_v, sem)
    ch.wait()
    cr.wait()
    ct.wait()

    lane = lax.iota(jnp.int32, _L)

    def group_body(g, carry):
      rows = g * _L + lane
      scores = jnp.zeros((_L,), jnp.float32)
      for d in range(half):
        re_col = jnp.full((_L,), d, jnp.int32)
        im_col = jnp.full((_L,), half + d, jnp.int32)
        rh = plsc.load_gather(hrow_v, [rows, re_col])
        ih = plsc.load_gather(hrow_v, [rows, im_col])
        rr = plsc.load_gather(rrow_v, [rows, re_col])
        ir = plsc.load_gather(rrow_v, [rows, im_col])
        rt = plsc.load_gather(trow_v, [rows, re_col])
        it = plsc.load_gather(trow_v, [rows, im_col])
        scores = scores + (rh * rr - ih * ir) * rt + (rh * ir + ih * rr) * it
      out_v[pl.ds(g * _L, _L)] = scores
      return carry

    lax.fori_loop(0, bpw // _L, group_body, 0)
    pltpu.sync_copy(out_v, out_hbm.at[pl.ds(base, bpw)])

  return sc_score


def kernel(sample, entity_embedding, relation_embedding):
  batch = sample.shape[0]
  dim = entity_embedding.shape[1]
  hidx = sample[:, 0]
  ridx = sample[:, 1]
  tidx = sample[:, 2]
  score = _make_sc_score(batch, dim)(
      hidx, ridx, tidx, entity_embedding, relation_embedding)
  return score.reshape(batch, 1)
